# baseline (speedup 1.0000x reference)
import functools

import jax
import jax.numpy as jnp
from jax import lax
from jax.experimental import pallas as pl
from jax.experimental.pallas import tpu as pltpu
from jax.experimental.pallas import tpu_sc as plsc

_F, _V, _D = 26, 100000, 32
_N = 64
_LANES = 16
_BL = 512
_NBF = _V // _BL
_REM = _V - _NBF * _BL
_VTAIL = _REM - 128


def _worker_count():
    try:
        info = plsc.get_sparse_core_info()
        return info.num_cores, info.num_subcores
    except Exception:
        return 2, 16


@functools.lru_cache(maxsize=None)
def _build_pack():
    nc, ns = _worker_count()
    nw = nc * ns
    ub_per_f = _NBF + 1
    units = _F * ub_per_f
    rows_per_f = _V * _D // 128
    mesh = plsc.VectorSubcoreMesh(core_axis_name="c", subcore_axis_name="s")

    @functools.partial(
        pl.kernel,
        mesh=mesh,
        compiler_params=pltpu.CompilerParams(
            use_tc_tiling_on_sc=True, needs_layout_passes=False),
        out_type=jax.ShapeDtypeStruct((_F * rows_per_f, 128), jnp.float32),
        scratch_types=[
            pltpu.VMEM((2, _D, _BL), jnp.float32),
            pltpu.VMEM((2, _BL * _D // 128, 128), jnp.float32),
            pltpu.VMEM((_D, _VTAIL), jnp.float32),
            pltpu.SemaphoreType.DMA,
            pltpu.SemaphoreType.DMA,
            pltpu.SemaphoreType.DMA,
            pltpu.SemaphoreType.DMA,
        ],
    )
    def pack_kernel(tab_fdv, tail_fdv, out_hbm, in_v, pk_v, tl_v,
                    si0, si1, so0, so1):
        sis = (si0, si1)
        sos = (so0, so1)
        wid = lax.axis_index("s") * nc + lax.axis_index("c")
        lane = lax.iota(jnp.int32, _LANES)

        def unit_fv(u):
            return u // ub_per_f, lax.rem(u, ub_per_f)

        def issue_in(u, b):
            f, j = unit_fv(u)
            nl = jnp.where(j < _NBF, _BL, 128)

            @pl.when(j < _NBF)
            def _():
                pltpu.async_copy(tab_fdv.at[f, :, pl.ds(j * _BL, _BL)],
                                 in_v.at[b], sis[b])

            @pl.when(j >= _NBF)
            def _():
                pltpu.async_copy(
                    tab_fdv.at[f, :, pl.ds(_NBF * _BL, 128)],
                    in_v.at[b, :, pl.ds(0, 128)], sis[b])
            return nl

        def wait_in(b, full):
            if full:
                pltpu.make_async_copy(
                    tab_fdv.at[0, :, pl.ds(0, _BL)], in_v.at[b],
                    sis[b]).wait()
            else:
                pltpu.make_async_copy(
                    tab_fdv.at[0, :, pl.ds(0, 128)],
                    in_v.at[b, :, pl.ds(0, 128)], sis[b]).wait()

        def transpose_block(b, nlanes):
            lane32 = lane * _D
            for d in range(_D):
                flat0 = lane32 + d
                col_d = lax.bitwise_and(flat0, 127)
                row_d = lax.shift_right_logical(flat0, 7)
                for k in range(nlanes // _LANES):
                    vals = in_v[b, d, pl.ds(k * _LANES, _LANES)]
                    row = row_d + (k * _LANES * _D) // 128
                    plsc.store_scatter(pk_v.at[b], [row, col_d], vals)

        def store_out(u, b, full):
            f, j = unit_fv(u)
            base = f * rows_per_f + j * (_BL * _D // 128)
            if full:
                pltpu.async_copy(pk_v.at[b],
                                 out_hbm.at[pl.ds(base, _BL * _D // 128)],
                                 sos[b])
            else:
                pltpu.async_copy(pk_v.at[b, pl.ds(0, _D)],
                                 out_hbm.at[pl.ds(base, _D)], sos[b])

        def wait_out(b, full):
            n = _BL * _D // 128 if full else _D
            pltpu.make_async_copy(pk_v.at[b, pl.ds(0, n)],
                                  out_hbm.at[pl.ds(0, n)], sos[b]).wait()

        nu = units // nw
        rem = units - nu * nw
        my_units = nu + jnp.where(wid < rem, 1, 0)

        @pl.when(my_units > 0)
        def _():
            issue_in(wid, 0)

        def loop(i, carry):
            u = wid + i * nw
            for b in range(2):
                @pl.when(lax.rem(i, 2) == b)
                def _():
                    f, j = unit_fv(u)

                    @pl.when(i + 1 < my_units)
                    def _():
                        issue_in(u + nw, 1 - b)

                    for full in (True, False):
                        cond = (j < _NBF) if full else (j >= _NBF)
                        @pl.when(cond)
                        def _():
                            wait_in(b, full)

                            @pl.when(i >= 2)
                            def _():
                                fp, jp = unit_fv(u - 2 * nw)
                                @pl.when(jp < _NBF)
                                def _():
                                    wait_out(b, True)
                                @pl.when(jp >= _NBF)
                                def _():
                                    wait_out(b, False)

                            transpose_block(b, _BL if full else 128)
                            store_out(u, b, full)
            return carry

        lax.fori_loop(0, my_units, loop, 0)

        def drain_last(t):
            i = my_units - t

            @pl.when(i >= 0)
            def _():
                _, jp = unit_fv(wid + i * nw)
                bb = lax.rem(i, 2)
                for b in range(2):
                    @pl.when(bb == b)
                    def _():
                        @pl.when(jp < _NBF)
                        def _():
                            wait_out(b, True)

                        @pl.when(jp >= _NBF)
                        def _():
                            wait_out(b, False)

        drain_last(2)
        drain_last(1)

        @pl.when(wid < _F)
        def _():
            f = wid
            pltpu.sync_copy(tail_fdv.at[f], tl_v)
            lane32 = lane * _D
            for d in range(_D):
                flat0 = lane32 + d
                col_d = lax.bitwise_and(flat0, 127)
                row_d = lax.shift_right_logical(flat0, 7)
                for k in range(_VTAIL // _LANES):
                    vals = tl_v[d, pl.ds(k * _LANES, _LANES)]
                    row = row_d + (k * _LANES * _D) // 128
                    plsc.store_scatter(pk_v.at[0], [row, col_d], vals)
            nrows = _VTAIL * _D // 128
            pltpu.sync_copy(
                pk_v.at[0, pl.ds(0, nrows)],
                out_hbm.at[pl.ds(f * rows_per_f + (_V - _VTAIL) * _D // 128,
                                 nrows)])

    return pack_kernel


@functools.lru_cache(maxsize=None)
def _build_main(b_, l_):
    nc, ns = _worker_count()
    nw = nc * ns
    b_per_w = b_ // nw
    bch = b_per_w // _N
    nch = bch * l_

    mesh = plsc.VectorSubcoreMesh(core_axis_name="c", subcore_axis_name="s")

    @functools.partial(
        pl.kernel,
        mesh=mesh,
        compiler_params=pltpu.CompilerParams(
            use_tc_tiling_on_sc=False, needs_layout_passes=False),
        out_type=jax.ShapeDtypeStruct((_D, l_, b_), jnp.float32),
        scratch_types=[
            pltpu.VMEM((3, _F, _N), jnp.int32),
            pltpu.VMEM((2, _F * _N, _D), jnp.float32),
            pltpu.VMEM((2, _D, _N), jnp.float32),
            pltpu.SemaphoreType.DMA,
            pltpu.SemaphoreType.DMA,
            pltpu.SemaphoreType.DMA,
            pltpu.SemaphoreType.DMA,
            pltpu.SemaphoreType.DMA,
            pltpu.SemaphoreType.DMA,
            pltpu.SemaphoreType.DMA,
        ],
    )
    def sc_kernel(phase_flb, tab_hbm, out_hbm,
                  idx_v, rows_v, out_v, i0, i1, i2, g0, g1, s0, s1):
        isem = (i0, i1, i2)
        gsem = (g0, g1)
        osem = (s0, s1)
        wid = lax.axis_index("s") * nc + lax.axis_index("c")
        b0w = wid * b_per_w
        lane = lax.iota(jnp.int32, _LANES)

        def chunk_lb(ch):
            return ch // bch, b0w + lax.rem(ch, bch) * _N

        def issue_idx(ch):
            ll, bb = chunk_lb(ch)
            k = lax.rem(ch, 3)
            for j in range(3):
                @pl.when(k == j)
                def _():
                    pltpu.async_copy(phase_flb.at[:, ll, pl.ds(bb, _N)],
                                     idx_v.at[j], isem[j])

        def wait_idx(ch):
            k = lax.rem(ch, 3)
            for j in range(3):
                @pl.when(k == j)
                def _():
                    pltpu.make_async_copy(
                        phase_flb.at[:, 0, pl.ds(0, _N)], idx_v.at[j],
                        isem[j]).wait()

        def fire(ch, b):
            wait_idx(ch)
            k = lax.rem(ch, 3)
            for j in range(3):
                @pl.when(k == j)
                def _():
                    for f in range(_F):
                        pltpu.async_copy(
                            tab_hbm.at[f].at[idx_v.at[j, f]],
                            rows_v.at[b, pl.ds(f * _N, _N)],
                            gsem[b])

            @pl.when(ch + 2 < nch)
            def _():
                issue_idx(ch + 2)

        def drain_gather(b):
            pltpu.make_async_copy(
                tab_hbm.at[0].at[pl.ds(0, _N * _F)], rows_v.at[b],
                gsem[b]).wait()

        def drain_out(b):
            pltpu.make_async_copy(
                out_v.at[b], out_hbm.at[:, 0, pl.ds(0, _N)], osem[b]).wait()

        def accum(b):
            def body(n, carry):
                lo = pl.ds(0, _LANES)
                hi = pl.ds(_LANES, _LANES)
                acc0 = rows_v[b, n, lo]
                acc1 = rows_v[b, n, hi]
                for f in range(1, _F):
                    acc0 = acc0 + rows_v[b, f * _N + n, lo]
                    acc1 = acc1 + rows_v[b, f * _N + n, hi]
                ncol = jnp.zeros((_LANES,), jnp.int32) + n
                plsc.store_scatter(out_v.at[b], [lane, ncol], acc0)
                plsc.store_scatter(out_v.at[b], [lane + _LANES, ncol], acc1)
                return carry
            lax.fori_loop(0, _N, body, 0)

        issue_idx(0)
        issue_idx(1)
        fire(0, 0)

        def outer(g, carry):
            for b in range(2):
                ch = g * 2 + b

                @pl.when(ch + 1 < nch)
                def _():
                    fire(ch + 1, 1 - b)

                drain_gather(b)

                @pl.when(ch >= 2)
                def _():
                    drain_out(b)

                accum(b)
                ll, bb = chunk_lb(ch)
                pltpu.async_copy(
                    out_v.at[b],
                    out_hbm.at[:, ll, pl.ds(bb, _N)],
                    osem[b])
            return carry

        lax.fori_loop(0, nch // 2, outer, 0)
        drain_out(0)
        drain_out(1)

    return sc_kernel


def kernel(phase, tables):
    phase = phase.astype(jnp.int32)
    b_, l_ = phase.shape[0], phase.shape[1]
    tab_fdv = jnp.transpose(tables, (0, 2, 1))
    tail_fdv = lax.slice_in_dim(tab_fdv, _V - _VTAIL, _V, axis=2)
    packed = _build_pack()(tab_fdv, tail_fdv)
    tab3 = packed.reshape(_F, _V, _D)
    phase_flb = jnp.transpose(phase, (2, 1, 0))
    out_dlb = _build_main(b_, l_)(phase_flb, tab3)
    return jnp.transpose(out_dlb, (2, 1, 0))


# speedup vs baseline: 1.2875x; 1.2875x over previous
import functools

import jax
import jax.numpy as jnp
from jax import lax
from jax.experimental import pallas as pl
from jax.experimental.pallas import tpu as pltpu
from jax.experimental.pallas import tpu_sc as plsc

_F, _V, _D = 26, 100000, 32
_N = 64
_LANES = 16


def _worker_count():
    try:
        info = plsc.get_sparse_core_info()
        return info.num_cores, info.num_subcores
    except Exception:
        return 2, 16


@functools.lru_cache(maxsize=None)
def _build_main(b_, l_):
    nc, ns = _worker_count()
    nw = nc * ns
    b_per_w = b_ // nw
    bch = b_per_w // _N
    nch = bch * l_

    mesh = plsc.VectorSubcoreMesh(core_axis_name="c", subcore_axis_name="s")

    @functools.partial(
        pl.kernel,
        mesh=mesh,
        compiler_params=pltpu.CompilerParams(
            use_tc_tiling_on_sc=False, needs_layout_passes=False),
        out_type=jax.ShapeDtypeStruct((_D, l_, b_), jnp.float32),
        scratch_types=[
            pltpu.VMEM((3, _F, _N), jnp.int32),
            pltpu.VMEM((2, _F * _N, _D), jnp.float32),
            pltpu.VMEM((2, _D, _N), jnp.float32),
            pltpu.SemaphoreType.DMA,
            pltpu.SemaphoreType.DMA,
            pltpu.SemaphoreType.DMA,
            pltpu.SemaphoreType.DMA,
            pltpu.SemaphoreType.DMA,
            pltpu.SemaphoreType.DMA,
            pltpu.SemaphoreType.DMA,
        ],
    )
    def sc_kernel(phase_flb, tab_hbm, out_hbm,
                  idx_v, rows_v, out_v, i0, i1, i2, g0, g1, s0, s1):
        isem = (i0, i1, i2)
        gsem = (g0, g1)
        osem = (s0, s1)
        wid = lax.axis_index("s") * nc + lax.axis_index("c")
        b0w = wid * b_per_w
        lane = lax.iota(jnp.int32, _LANES)

        def chunk_lb(ch):
            return ch // bch, b0w + lax.rem(ch, bch) * _N

        def issue_idx(ch):
            ll, bb = chunk_lb(ch)
            k = lax.rem(ch, 3)
            for j in range(3):
                @pl.when(k == j)
                def _():
                    pltpu.async_copy(phase_flb.at[:, ll, pl.ds(bb, _N)],
                                     idx_v.at[j], isem[j])

        def wait_idx(ch):
            k = lax.rem(ch, 3)
            for j in range(3):
                @pl.when(k == j)
                def _():
                    pltpu.make_async_copy(
                        phase_flb.at[:, 0, pl.ds(0, _N)], idx_v.at[j],
                        isem[j]).wait()

        def fire(ch, b):
            wait_idx(ch)
            k = lax.rem(ch, 3)
            for j in range(3):
                @pl.when(k == j)
                def _():
                    for f in range(_F):
                        pltpu.async_copy(
                            tab_hbm.at[f].at[idx_v.at[j, f]],
                            rows_v.at[b, pl.ds(f * _N, _N)],
                            gsem[b])

            @pl.when(ch + 2 < nch)
            def _():
                issue_idx(ch + 2)

        def drain_gather(b):
            pltpu.make_async_copy(
                tab_hbm.at[0].at[pl.ds(0, _N * _F)], rows_v.at[b],
                gsem[b]).wait()

        def drain_out(b):
            pltpu.make_async_copy(
                out_v.at[b], out_hbm.at[:, 0, pl.ds(0, _N)], osem[b]).wait()

        def accum(b):
            def body(n, carry):
                lo = pl.ds(0, _LANES)
                hi = pl.ds(_LANES, _LANES)
                acc0 = rows_v[b, n, lo]
                acc1 = rows_v[b, n, hi]
                for f in range(1, _F):
                    acc0 = acc0 + rows_v[b, f * _N + n, lo]
                    acc1 = acc1 + rows_v[b, f * _N + n, hi]
                ncol = jnp.zeros((_LANES,), jnp.int32) + n
                plsc.store_scatter(out_v.at[b], [lane, ncol], acc0)
                plsc.store_scatter(out_v.at[b], [lane + _LANES, ncol], acc1)
                return carry
            lax.fori_loop(0, _N, body, 0)

        issue_idx(0)
        issue_idx(1)
        fire(0, 0)

        def outer(g, carry):
            for b in range(2):
                ch = g * 2 + b

                @pl.when(ch + 1 < nch)
                def _():
                    fire(ch + 1, 1 - b)

                drain_gather(b)

                @pl.when(ch >= 2)
                def _():
                    drain_out(b)

                accum(b)
                ll, bb = chunk_lb(ch)
                pltpu.async_copy(
                    out_v.at[b],
                    out_hbm.at[:, ll, pl.ds(bb, _N)],
                    osem[b])
            return carry

        lax.fori_loop(0, nch // 2, outer, 0)
        drain_out(0)
        drain_out(1)

    return sc_kernel


def kernel(phase, tables):
    phase = phase.astype(jnp.int32)
    b_, l_ = phase.shape[0], phase.shape[1]
    phase_flb = jnp.transpose(phase, (2, 1, 0))
    out_dlb = _build_main(b_, l_)(phase_flb, tables)
    return jnp.transpose(out_dlb, (2, 1, 0))
